# baseline (speedup 1.0000x reference)
import numpy as np
import jax
import jax.numpy as jnp
from jax.experimental import pallas as pl
from jax.experimental.pallas import tpu as pltpu

_Nr = 1000
_Ng = 32
_C = 200
_B = 8
_T = 4032

_FG_IOU = 0.7
_BG_IOU = 0.01


def _body(rr3_ref, segc_ref, tri32_ref, cls_ref, pts_ref, out_ref, acc_ref):
    j = pl.program_id(0)

    @pl.when(j == 0)
    def _init():
        x = pts_ref[0:8]
        g = pts_ref[8:16]
        m = pts_ref[16:24]
        t = (g > 0.5).astype(jnp.float32)
        ax = jnp.abs(x)
        l1p = jnp.log1p(jnp.exp(-ax))
        ls_pos = jnp.minimum(x, 0.0) - l1p
        ls_neg = jnp.minimum(-x, 0.0) - l1p
        ce_f = -(t * ls_pos + (1.0 - t) * ls_neg)
        p = 1.0 / (1.0 + jnp.exp(-x))
        p_t = p * t + (1.0 - p) * (1.0 - t)
        q = 1.0 - p_t
        alpha_t = 0.25 * t + 0.75 * (1.0 - t)
        fl = alpha_t * ce_f * q * q
        acc_ref[2] = jnp.sum(fl * m)
        posm = t * m
        acc_ref[4] = jnp.sum(posm)

        lp = pts_ref[24:32]
        rp = pts_ref[32:40]
        lg = pts_ref[40:48]
        rg = pts_ref[48:56]
        intsctk = jnp.minimum(rp, rg) + jnp.minimum(lp, lg)
        unionk = (lp + rp) + (lg + rg) - intsctk
        iouk = intsctk / jnp.maximum(unionk, 1e-8)
        len_c = jnp.maximum(lp, lg) + jnp.maximum(rp, rg)
        rho = 0.5 * (rp - lp - rg + lg)
        rr_ = rho / jnp.maximum(len_c, 1e-8)
        dl = 1.0 - iouk + rr_ * rr_
        acc_ref[3] = jnp.sum(dl * posm)

        acc_ref[0] = 0.0
        acc_ref[1] = 0.0

    rl = rr3_ref[pl.ds(j, 1)]
    rr = rr3_ref[pl.ds(j + 8, 1)]
    sc = rr3_ref[pl.ds(j + 16, 1)]
    segc = segc_ref[0]
    gl = segc[:, 0:1]
    gr = segc[:, 1:2]
    labc = segc[:, 2:3]
    min_l = jnp.minimum(gl, rl)
    max_l = jnp.maximum(gl, rl)
    min_r = jnp.minimum(gr, rr)
    max_r = jnp.maximum(gr, rr)
    mat = (min_r - max_l) / (max_r - min_l)
    ious = jnp.max(mat, axis=0, keepdims=True)
    ismax = mat >= ious
    ismax_bf = ismax.astype(jnp.bfloat16)
    cnt = jnp.dot(tri32_ref[...], ismax_bf,
                  preferred_element_type=jnp.float32)
    posf = (ious > _FG_IOU).astype(jnp.float32)
    npos = jnp.sum(posf)
    G = jnp.where(ismax & (cnt < 1.5), posf, 0.0)
    bgf = jnp.where((ious < _BG_IOU) & (sc > 0.0), 1.0, 0.0)
    cum = bgf
    for k in (1, 2, 4, 8, 16, 32, 64, 128, 256, 512):
        cum = cum + jnp.concatenate(
            [jnp.zeros((1, k), jnp.float32), cum[:, :-k]], axis=1)
    bg_sel = bgf * (cum < npos + 0.5).astype(jnp.float32)
    sel = jnp.maximum(posf, bg_sel)

    cls = cls_ref[0]
    rowmax = jnp.max(cls, axis=1, keepdims=True)
    esum = jnp.sum(jnp.exp(cls - rowmax), axis=1, keepdims=True)
    lse = rowmax + jnp.log(esum)
    s_lse = jnp.dot(sel, lse, preferred_element_type=jnp.float32)[0, 0]
    s_bg = jnp.dot(bg_sel, cls[:, 0:1],
                   preferred_element_type=jnp.float32)[0, 0]
    GC = jnp.dot(G, cls, preferred_element_type=jnp.float32)
    cio = jax.lax.broadcasted_iota(jnp.int32, (_Ng, _C), 1)
    s_pos = jnp.sum(jnp.where(cio == labc.astype(jnp.int32), GC, 0.0))
    acc_ref[0] = acc_ref[0] + (s_lse - s_bg - s_pos)
    acc_ref[1] = acc_ref[1] + jnp.sum(sel)

    @pl.when(j == _B - 1)
    def _fin():
        norm = 90.0 + 0.1 * jnp.maximum(acc_ref[4], 1.0)
        out_ref[0, 0] = (acc_ref[2] + acc_ref[3]) / norm + acc_ref[0] / acc_ref[1]


_TRI32 = np.tri(_Ng, dtype=np.float32).astype(jnp.bfloat16)


def kernel(fpn_masks, out_cls_logits, out_offsets, out_rois, out_scores,
           out_roimask, cls_log, gt_cls, gt_offsets, gt_segments,
           segments_label, segments_mask):
    f32 = jnp.float32
    tri32 = jnp.asarray(_TRI32)
    rr3 = jnp.concatenate(
        [out_rois[:, :, 1], out_rois[:, :, 2], out_scores], axis=0)
    segc = jnp.stack(
        [gt_segments[:, :, 0], gt_segments[:, :, 1],
         segments_label.astype(f32)], axis=2)
    pts = jnp.concatenate(
        [out_cls_logits, gt_cls.astype(f32), fpn_masks.astype(f32),
         out_offsets[:, :, 0], out_offsets[:, :, 1],
         gt_offsets[:, :, 0], gt_offsets[:, :, 1]], axis=0)

    out = pl.pallas_call(
        _body,
        grid=(_B,),
        in_specs=[
            pl.BlockSpec((3 * _B, _Nr), lambda j: (0, 0)),
            pl.BlockSpec((1, _Ng, 3), lambda j: (j, 0, 0)),
            pl.BlockSpec((_Ng, _Ng), lambda j: (0, 0)),
            pl.BlockSpec((1, _Nr, _C), lambda j: (j, 0, 0)),
            pl.BlockSpec((7 * _B, _T), lambda j: (0, 0)),
        ],
        out_specs=pl.BlockSpec((1, 1), lambda j: (0, 0), memory_space=pltpu.SMEM),
        out_shape=jax.ShapeDtypeStruct((1, 1), f32),
        scratch_shapes=[pltpu.SMEM((8,), f32)],
    )(rr3, segc, tri32, cls_log, pts)
    return out[0, 0]


# speedup vs baseline: 1.2119x; 1.2119x over previous
import numpy as np
import jax
import jax.numpy as jnp
from jax.experimental import pallas as pl
from jax.experimental.pallas import tpu as pltpu

_Nr = 1000
_Ng = 32
_C = 200
_B = 8
_T = 4032

_FG_IOU = 0.7
_BG_IOU = 0.01


def _body(rr3_ref, segc_ref, tri32_ref, cls_ref, pts_ref, out_ref, acc_ref):
    j = pl.program_id(0)

    @pl.when(j == 0)
    def _init():
        x = pts_ref[0:8]
        g = pts_ref[8:16]
        m = pts_ref[16:24]
        t = (g > 0.5).astype(jnp.float32)
        ax = jnp.abs(x)
        l1p = jnp.log1p(jnp.exp(-ax))
        ls_pos = jnp.minimum(x, 0.0) - l1p
        ls_neg = jnp.minimum(-x, 0.0) - l1p
        ce_f = -(t * ls_pos + (1.0 - t) * ls_neg)
        p = 1.0 / (1.0 + jnp.exp(-x))
        p_t = p * t + (1.0 - p) * (1.0 - t)
        q = 1.0 - p_t
        alpha_t = 0.25 * t + 0.75 * (1.0 - t)
        fl = alpha_t * ce_f * q * q
        acc_ref[2] = jnp.sum(fl * m)
        posm = t * m
        acc_ref[4] = jnp.sum(posm)

        lp = pts_ref[24:32]
        rp = pts_ref[32:40]
        lg = pts_ref[40:48]
        rg = pts_ref[48:56]
        intsctk = jnp.minimum(rp, rg) + jnp.minimum(lp, lg)
        unionk = (lp + rp) + (lg + rg) - intsctk
        iouk = intsctk / jnp.maximum(unionk, 1e-8)
        len_c = jnp.maximum(lp, lg) + jnp.maximum(rp, rg)
        rho = 0.5 * (rp - lp - rg + lg)
        rr_ = rho / jnp.maximum(len_c, 1e-8)
        dl = 1.0 - iouk + rr_ * rr_
        acc_ref[3] = jnp.sum(dl * posm)

        acc_ref[0] = 0.0
        acc_ref[1] = 0.0

    rl = rr3_ref[pl.ds(j, 1)]
    rr = rr3_ref[pl.ds(j + 8, 1)]
    sc = rr3_ref[pl.ds(j + 16, 1)]
    segc = segc_ref[0]
    gl = segc[:, 0:1]
    gr = segc[:, 1:2]
    labc = segc[:, 2:3]
    min_l = jnp.minimum(gl, rl)
    max_l = jnp.maximum(gl, rl)
    min_r = jnp.minimum(gr, rr)
    max_r = jnp.maximum(gr, rr)
    mat = (min_r - max_l) / (max_r - min_l)
    ious = jnp.max(mat, axis=0, keepdims=True)
    ismax = mat >= ious
    ismax_bf = ismax.astype(jnp.bfloat16)
    cnt = jnp.dot(tri32_ref[...], ismax_bf,
                  preferred_element_type=jnp.float32)
    posf = (ious > _FG_IOU).astype(jnp.float32)
    npos = jnp.sum(posf)
    G = jnp.where(ismax & (cnt < 1.5), posf, 0.0)
    bgf = jnp.where((ious < _BG_IOU) & (sc > 0.0), 1.0, 0.0)
    cum = bgf
    for k in (1, 2, 4, 8, 16, 32, 64, 128, 256, 512):
        cum = cum + jnp.concatenate(
            [jnp.zeros((1, k), jnp.float32), cum[:, :-k]], axis=1)
    bg_sel = bgf * (cum < npos + 0.5).astype(jnp.float32)
    sel = jnp.maximum(posf, bg_sel)

    cls = cls_ref[0].astype(jnp.float32)
    rowmax = jnp.max(cls, axis=1, keepdims=True)
    esum = jnp.sum(jnp.exp(cls - rowmax), axis=1, keepdims=True)
    lse = rowmax + jnp.log(esum)
    s_lse = jnp.dot(sel, lse, preferred_element_type=jnp.float32)[0, 0]
    s_bg = jnp.dot(bg_sel, cls[:, 0:1],
                   preferred_element_type=jnp.float32)[0, 0]
    GC = jnp.dot(G, cls, preferred_element_type=jnp.float32)
    cio = jax.lax.broadcasted_iota(jnp.int32, (_Ng, _C), 1)
    s_pos = jnp.sum(jnp.where(cio == labc.astype(jnp.int32), GC, 0.0))
    acc_ref[0] = acc_ref[0] + (s_lse - s_bg - s_pos)
    acc_ref[1] = acc_ref[1] + jnp.sum(sel)

    @pl.when(j == _B - 1)
    def _fin():
        norm = 90.0 + 0.1 * jnp.maximum(acc_ref[4], 1.0)
        out_ref[0, 0] = (acc_ref[2] + acc_ref[3]) / norm + acc_ref[0] / acc_ref[1]


_TRI32 = np.tri(_Ng, dtype=np.float32).astype(jnp.bfloat16)


def kernel(fpn_masks, out_cls_logits, out_offsets, out_rois, out_scores,
           out_roimask, cls_log, gt_cls, gt_offsets, gt_segments,
           segments_label, segments_mask):
    f32 = jnp.float32
    tri32 = jnp.asarray(_TRI32)
    clsh = cls_log.astype(jnp.bfloat16)
    rr3 = jnp.concatenate(
        [out_rois[:, :, 1], out_rois[:, :, 2], out_scores], axis=0)
    segc = jnp.stack(
        [gt_segments[:, :, 0], gt_segments[:, :, 1],
         segments_label.astype(f32)], axis=2)
    pts = jnp.concatenate(
        [out_cls_logits, gt_cls.astype(f32), fpn_masks.astype(f32),
         out_offsets[:, :, 0], out_offsets[:, :, 1],
         gt_offsets[:, :, 0], gt_offsets[:, :, 1]], axis=0)

    out = pl.pallas_call(
        _body,
        grid=(_B,),
        in_specs=[
            pl.BlockSpec((3 * _B, _Nr), lambda j: (0, 0)),
            pl.BlockSpec((1, _Ng, 3), lambda j: (j, 0, 0)),
            pl.BlockSpec((_Ng, _Ng), lambda j: (0, 0)),
            pl.BlockSpec((1, _Nr, _C), lambda j: (j, 0, 0)),
            pl.BlockSpec((7 * _B, _T), lambda j: (0, 0)),
        ],
        out_specs=pl.BlockSpec((1, 1), lambda j: (0, 0), memory_space=pltpu.SMEM),
        out_shape=jax.ShapeDtypeStruct((1, 1), f32),
        scratch_shapes=[pltpu.SMEM((8,), f32)],
    )(rr3, segc, tri32, clsh, pts)
    return out[0, 0]
